# Initial kernel scaffold; baseline (speedup 1.0000x reference)
import jax
import jax.numpy as jnp
from jax.experimental import pallas as pl


def kernel(x, c1w, c1b, c2w, c2b, c3w, c3b, fc1_wt, fc1_b, fc2_wt, fc2_b):
    raise NotImplementedError("write your pallas kernel here")


# speedup vs baseline: 6.6498x; 6.6498x over previous
import functools

import jax
import jax.numpy as jnp
from jax.experimental import pallas as pl
from jax.experimental.pallas import tpu as pltpu

G = 16
NGROUPS = 256 // G

R1 = 72
R2 = 40
R3 = 24


def _pool2x2(v, sel_ref):
    rows, cols = v.shape
    vr = jnp.max(v.reshape(rows // 2, 2, cols), axis=1)
    m = jnp.maximum(vr, jnp.roll(vr, shift=-1, axis=1))
    return jnp.dot(m, sel_ref[...],
                   preferred_element_type=jnp.float32)


def _conv_net_kernel(x_ref, b1_ref, b2_ref, b3_ref, a1_ref, a2_ref, a3_ref,
                     sel_ref, o_ref, sx_ref, s2_ref, s3_ref):
    f32 = jnp.float32

    sx_ref[0:G * R1, :] = x_ref[...]
    M1 = G * R1
    y1 = jnp.dot(sx_ref[0:M1, :], b1_ref[0], preferred_element_type=f32)
    for dy in range(1, 5):
        y1 = y1 + jnp.dot(sx_ref[dy:dy + M1, :], b1_ref[dy],
                          preferred_element_type=f32)
    y1 = jnp.maximum(y1 + a1_ref[...], 0.0)

    p1 = _pool2x2(y1, sel_ref)

    s2_ref[...] = jnp.zeros_like(s2_ref)
    for g in range(G):
        s2_ref[g * R2 + 2:g * R2 + 34, :] = p1[g * 36:g * 36 + 32, :]

    M2 = G * R2
    y2 = jnp.dot(s2_ref[0:M2, :], b2_ref[0], preferred_element_type=f32)
    for dy in range(1, 5):
        y2 = y2 + jnp.dot(s2_ref[dy:dy + M2, :], b2_ref[dy],
                          preferred_element_type=f32)
    y2 = jnp.maximum(y2 + a2_ref[...], 0.0)

    p2 = _pool2x2(y2, sel_ref)

    s3_ref[...] = jnp.zeros_like(s3_ref)
    for g in range(G):
        s3_ref[g * R3 + 2:g * R3 + 18, :] = p2[g * 20:g * 20 + 16, :]

    M3 = G * R3
    y3 = jnp.dot(s3_ref[0:M3, :], b3_ref[0], preferred_element_type=f32)
    for dy in range(1, 5):
        y3 = y3 + jnp.dot(s3_ref[dy:dy + M3, :], b3_ref[dy],
                          preferred_element_type=f32)
    y3 = jnp.maximum(y3 + a3_ref[...], 0.0)

    for g in range(G):
        o_ref[g * 16:(g + 1) * 16, :] = y3[g * R3:g * R3 + 16, :]


def _fc_kernel(f_ref, w_ref, b1_ref, w2_ref, b2_ref, o_ref, acc_ref):
    j = pl.program_id(1)

    @pl.when(j == 0)
    def _():
        acc_ref[...] = jnp.zeros_like(acc_ref)

    w = w_ref[...].reshape(1920, 128)
    acc_ref[...] += jnp.dot(f_ref[...], w, preferred_element_type=jnp.float32)

    @pl.when(j == pl.num_programs(1) - 1)
    def _():
        h = acc_ref[...] + b1_ref[...]
        o_ref[...] = jnp.dot(h, w2_ref[...],
                             preferred_element_type=jnp.float32) + b2_ref[...]


def _make_bands(w, cout, cin, width, dtype=jnp.float32):
    w4 = w.reshape(cout, 5, 5, cin)
    eyes = jnp.stack([jnp.eye(width, width, 2 - e, dtype=dtype)
                      for e in range(5)])
    band = jnp.einsum('odec,eix->dciox', w4, eyes)
    return band.reshape(5, cin * width, cout * width)


def kernel(x, c1w, c1b, c2w, c2b, c3w, c3b, fc1_wt, fc1_b, fc2_wt, fc2_b):
    x = x.astype(jnp.float32)
    B = x.shape[0]

    xp = jnp.pad(x.reshape(B, 64, 64), ((0, 0), (2, 6), (0, 0)))
    xp = xp.reshape(B * R1, 64)

    band1 = _make_bands(c1w, 8, 1, 64)
    band2 = _make_bands(c2w, 16, 8, 32)
    band3 = _make_bands(c3w, 120, 16, 16)
    a1 = jnp.repeat(c1b.reshape(-1), 64).reshape(1, 512)
    a2 = jnp.repeat(c2b.reshape(-1), 32).reshape(1, 512)
    a3 = jnp.repeat(c3b.reshape(-1), 16).reshape(1, 1920)
    sel = jnp.kron(jnp.eye(256, dtype=jnp.float32),
                   jnp.array([[1.0], [0.0]], jnp.float32))

    feat = pl.pallas_call(
        _conv_net_kernel,
        out_shape=jax.ShapeDtypeStruct((B * 16, 1920), jnp.float32),
        grid_spec=pltpu.PrefetchScalarGridSpec(
            num_scalar_prefetch=0,
            grid=(B // G,),
            in_specs=[
                pl.BlockSpec((G * R1, 64), lambda i: (i, 0)),
                pl.BlockSpec(band1.shape, lambda i: (0, 0, 0)),
                pl.BlockSpec(band2.shape, lambda i: (0, 0, 0)),
                pl.BlockSpec(band3.shape, lambda i: (0, 0, 0)),
                pl.BlockSpec(a1.shape, lambda i: (0, 0)),
                pl.BlockSpec(a2.shape, lambda i: (0, 0)),
                pl.BlockSpec(a3.shape, lambda i: (0, 0)),
                pl.BlockSpec(sel.shape, lambda i: (0, 0)),
            ],
            out_specs=pl.BlockSpec((G * 16, 1920), lambda i: (i, 0)),
            scratch_shapes=[
                pltpu.VMEM((G * R1 + 8, 64), jnp.float32),
                pltpu.VMEM((G * R2 + 16, 256), jnp.float32),
                pltpu.VMEM((G * R3 + 8, 256), jnp.float32),
            ],
        ),
        compiler_params=pltpu.CompilerParams(
            dimension_semantics=("parallel",)),
    )(xp, band1, band2, band3, a1, a2, a3, sel)

    f2 = feat.reshape(B, 16 * 1920)
    w3 = fc1_wt.reshape(120, 256, 128)

    out = pl.pallas_call(
        _fc_kernel,
        out_shape=jax.ShapeDtypeStruct((B, 7), jnp.float32),
        grid_spec=pltpu.PrefetchScalarGridSpec(
            num_scalar_prefetch=0,
            grid=(2, 16),
            in_specs=[
                pl.BlockSpec((B // 2, 1920), lambda i, j: (i, j)),
                pl.BlockSpec((120, 16, 128), lambda i, j: (0, j, 0)),
                pl.BlockSpec((1, 128), lambda i, j: (0, 0)),
                pl.BlockSpec((128, 7), lambda i, j: (0, 0)),
                pl.BlockSpec((1, 7), lambda i, j: (0, 0)),
            ],
            out_specs=pl.BlockSpec((B // 2, 7), lambda i, j: (i, 0)),
            scratch_shapes=[pltpu.VMEM((B // 2, 128), jnp.float32)],
        ),
        compiler_params=pltpu.CompilerParams(
            dimension_semantics=("parallel", "arbitrary")),
    )(f2, w3, fc1_b, fc2_wt, fc2_b)
    return out


# speedup vs baseline: 9.2514x; 1.3912x over previous
import jax
import jax.numpy as jnp
from jax.experimental import pallas as pl
from jax.experimental.pallas import tpu as pltpu

G = 16
RPI = 24
M = G * RPI
SLOP = 8


def _row_mask(like):
    r = jax.lax.broadcasted_iota(jnp.int32, (like.shape[0], 1), 0) % RPI
    keep = (r >= 4) & (r < 20)
    return jnp.where(keep, like, 0.0)


def _conv_class(slices, band_ref, bias_ref):
    acc = jnp.dot(slices[0], band_ref[0], preferred_element_type=jnp.float32)
    for dy in range(1, 5):
        acc = acc + jnp.dot(slices[dy], band_ref[dy],
                            preferred_element_type=jnp.float32)
    return jnp.maximum(acc + bias_ref[...], 0.0)


def _conv_net_kernel(x_ref, b1_ref, b2_ref, b3_ref, a1_ref, a2_ref, a3_ref,
                     o_ref, s2a_ref, s2b_ref, s3_ref):
    def x_slice(c, dy):
        v = c + dy - 2
        s = v % 4
        off = (v - s) // 4
        return x_ref[s, 0, 1 + off:385 + off, :]

    y1 = [None] * 4
    for c in range(4):
        y1[c] = _conv_class([x_slice(c, dy) for dy in range(5)],
                            b1_ref, a1_ref)

    for r, s2_ref in ((0, s2a_ref), (1, s2b_ref)):
        p = jnp.maximum(y1[2 * r], y1[2 * r + 1])
        p = jnp.maximum(p[:, 0:256], p[:, 256:512])
        s2_ref[1:385, :] = _row_mask(p)

    def s2_slice(c, dy):
        v = c + dy - 2
        s = v % 2
        off = (v - s) // 2
        ref = s2a_ref if s == 0 else s2b_ref
        return ref[1 + off:385 + off, :]

    y2 = [None] * 2
    for c in range(2):
        y2[c] = _conv_class([s2_slice(c, dy) for dy in range(5)],
                            b2_ref, a2_ref)

    p2 = jnp.maximum(y2[0], y2[1])
    p2 = jnp.maximum(p2[:, 0:256], p2[:, 256:512])
    s3_ref[2:386, :] = _row_mask(p2)

    y3 = _conv_class([s3_ref[dy:dy + 384, :] for dy in range(5)],
                     b3_ref, a3_ref)

    for g in range(G):
        o_ref[g * 16:(g + 1) * 16, :] = y3[g * RPI + 4:g * RPI + 20, :]


def _fc_kernel(f_ref, w_ref, b1_ref, w2_ref, b2_ref, o_ref, acc_ref):
    j = pl.program_id(1)

    @pl.when(j == 0)
    def _():
        acc_ref[...] = jnp.zeros_like(acc_ref)

    w = w_ref[...].reshape(1920, 128)
    acc_ref[...] += jnp.dot(f_ref[...], w, preferred_element_type=jnp.float32)

    @pl.when(j == pl.num_programs(1) - 1)
    def _():
        h = acc_ref[...] + b1_ref[...]
        o_ref[...] = jnp.dot(h, w2_ref[...],
                             preferred_element_type=jnp.float32) + b2_ref[...]


def _make_bands(w, cout, cin, width, split_x=True):
    f32 = jnp.float32
    w4 = w.reshape(cout, 5, 5, cin).astype(f32)
    eyes = jnp.stack([jnp.eye(width, width, 2 - e, dtype=f32)
                      for e in range(5)])
    band = jnp.einsum('odec,eix->dciox', w4, eyes)
    band = band.reshape(5, cin * width, cout, width)
    if split_x:
        band = band.reshape(5, cin * width, cout, width // 2, 2)
        band = band.transpose(0, 1, 4, 2, 3)
    return band.reshape(5, cin * width, cout * width)


def kernel(x, c1w, c1b, c2w, c2b, c3w, c3b, fc1_wt, fc1_b, fc2_wt, fc2_b):
    f32 = jnp.float32
    x = x.astype(f32)
    B = x.shape[0]
    ngrp = B // G

    x4 = x.reshape(B, 16, 4, 64).transpose(2, 0, 1, 3)
    x4 = jnp.pad(x4, ((0, 0), (0, 0), (5, 3), (0, 0)))
    x4 = x4.reshape(4, ngrp, G * RPI, 64)
    x4 = jnp.pad(x4, ((0, 0), (0, 0), (0, SLOP), (0, 0)))

    band1 = _make_bands(c1w, 8, 1, 64)
    band2 = _make_bands(c2w, 16, 8, 32)
    band3 = _make_bands(c3w, 120, 16, 16, split_x=False)

    def tile_bias(b, width, split_x=True):
        t = jnp.repeat(b.reshape(-1), width // (2 if split_x else 1))
        if split_x:
            t = jnp.tile(t, (2,))
        return t.reshape(1, -1).astype(f32)

    a1 = tile_bias(c1b, 64)
    a2 = tile_bias(c2b, 32)
    a3 = tile_bias(c3b, 16, split_x=False)

    feat = pl.pallas_call(
        _conv_net_kernel,
        out_shape=jax.ShapeDtypeStruct((B * 16, 1920), f32),
        grid_spec=pltpu.PrefetchScalarGridSpec(
            num_scalar_prefetch=0,
            grid=(ngrp,),
            in_specs=[
                pl.BlockSpec((4, 1, G * RPI + SLOP, 64),
                             lambda i: (0, i, 0, 0)),
                pl.BlockSpec(band1.shape, lambda i: (0, 0, 0)),
                pl.BlockSpec(band2.shape, lambda i: (0, 0, 0)),
                pl.BlockSpec(band3.shape, lambda i: (0, 0, 0)),
                pl.BlockSpec(a1.shape, lambda i: (0, 0)),
                pl.BlockSpec(a2.shape, lambda i: (0, 0)),
                pl.BlockSpec(a3.shape, lambda i: (0, 0)),
            ],
            out_specs=pl.BlockSpec((G * 16, 1920), lambda i: (i, 0)),
            scratch_shapes=[
                pltpu.VMEM((G * RPI + SLOP, 256), f32),
                pltpu.VMEM((G * RPI + SLOP, 256), f32),
                pltpu.VMEM((G * RPI + SLOP, 256), f32),
            ],
        ),
        compiler_params=pltpu.CompilerParams(
            dimension_semantics=("parallel",)),
    )(x4, band1, band2, band3, a1, a2, a3)

    f2 = feat.reshape(B, 16 * 1920)
    w3 = fc1_wt.reshape(120, 256, 128)

    out = pl.pallas_call(
        _fc_kernel,
        out_shape=jax.ShapeDtypeStruct((B, 7), f32),
        grid_spec=pltpu.PrefetchScalarGridSpec(
            num_scalar_prefetch=0,
            grid=(2, 16),
            in_specs=[
                pl.BlockSpec((B // 2, 1920), lambda i, j: (i, j)),
                pl.BlockSpec((120, 16, 128), lambda i, j: (0, j, 0)),
                pl.BlockSpec((1, 128), lambda i, j: (0, 0)),
                pl.BlockSpec((128, 7), lambda i, j: (0, 0)),
                pl.BlockSpec((1, 7), lambda i, j: (0, 0)),
            ],
            out_specs=pl.BlockSpec((B // 2, 7), lambda i, j: (i, 0)),
            scratch_shapes=[pltpu.VMEM((B // 2, 128), f32)],
        ),
        compiler_params=pltpu.CompilerParams(
            dimension_semantics=("parallel", "arbitrary")),
    )(f2, w3, fc1_b, fc2_wt, fc2_b)
    return out


# speedup vs baseline: 11.2712x; 1.2183x over previous
import jax
import jax.numpy as jnp
from jax.experimental import pallas as pl
from jax.experimental.pallas import tpu as pltpu

G = 32
RPI = 24
M = G * RPI
SLOP = 8
BF = jnp.bfloat16


def _row_mask(like):
    r = jax.lax.broadcasted_iota(jnp.int32, (like.shape[0], 1), 0) % RPI
    keep = (r >= 4) & (r < 20)
    return jnp.where(keep, like, 0.0)


def _conv_class(slices, band_ref, bias_ref):
    acc = jnp.dot(slices[0], band_ref[0], preferred_element_type=jnp.float32)
    for dy in range(1, 5):
        acc = acc + jnp.dot(slices[dy], band_ref[dy],
                            preferred_element_type=jnp.float32)
    return jnp.maximum(acc + bias_ref[...], 0.0)


def _conv_net_kernel(x_ref, b1_ref, b2_ref, b3_ref, a1_ref, a2_ref, a3_ref,
                     o_ref, s2a_ref, s2b_ref, s3_ref):
    def x_slice(c, dy):
        v = c + dy - 2
        s = v % 4
        off = (v - s) // 4
        return x_ref[s, 0, 1 + off:1 + off + M, :]

    y1 = [None] * 4
    for c in range(4):
        y1[c] = _conv_class([x_slice(c, dy) for dy in range(5)],
                            b1_ref, a1_ref)

    for r, s2_ref in ((0, s2a_ref), (1, s2b_ref)):
        p = jnp.maximum(y1[2 * r], y1[2 * r + 1])
        p = jnp.maximum(p[:, 0:256], p[:, 256:512])
        s2_ref[1:1 + M, :] = _row_mask(p).astype(BF)

    def s2_slice(c, dy):
        v = c + dy - 2
        s = v % 2
        off = (v - s) // 2
        ref = s2a_ref if s == 0 else s2b_ref
        return ref[1 + off:1 + off + M, :]

    y2 = [None] * 2
    for c in range(2):
        y2[c] = _conv_class([s2_slice(c, dy) for dy in range(5)],
                            b2_ref, a2_ref)

    p2 = jnp.maximum(y2[0], y2[1])
    p2 = jnp.maximum(p2[:, 0:256], p2[:, 256:512])
    s3_ref[2:2 + M, :] = _row_mask(p2).astype(BF)

    y3 = _conv_class([s3_ref[dy:dy + M, :] for dy in range(5)],
                     b3_ref, a3_ref)
    y3 = y3.astype(BF)

    for g in range(G):
        o_ref[g * 16:(g + 1) * 16, :] = y3[g * RPI + 4:g * RPI + 20, :]


def _fc_kernel(f_ref, w_ref, b1_ref, w2_ref, b2_ref, o_ref, acc_ref):
    j = pl.program_id(0)

    @pl.when(j == 0)
    def _():
        acc_ref[...] = jnp.zeros_like(acc_ref)

    w = w_ref[...].reshape(1920, 128).astype(BF)
    acc_ref[...] += jnp.dot(f_ref[...], w, preferred_element_type=jnp.float32)

    @pl.when(j == pl.num_programs(0) - 1)
    def _():
        h = acc_ref[...] + b1_ref[...]
        o_ref[...] = jnp.dot(h, w2_ref[...],
                             preferred_element_type=jnp.float32) + b2_ref[...]


def _make_bands(w, cout, cin, width, split_x=True):
    f32 = jnp.float32
    w4 = w.reshape(cout, 5, 5, cin).astype(f32)
    eyes = jnp.stack([jnp.eye(width, width, 2 - e, dtype=f32)
                      for e in range(5)])
    band = jnp.einsum('odec,eix->dciox', w4, eyes)
    band = band.reshape(5, cin * width, cout, width)
    if split_x:
        band = band.reshape(5, cin * width, cout, width // 2, 2)
        band = band.transpose(0, 1, 4, 2, 3)
    return band.reshape(5, cin * width, cout * width).astype(BF)


def kernel(x, c1w, c1b, c2w, c2b, c3w, c3b, fc1_wt, fc1_b, fc2_wt, fc2_b):
    f32 = jnp.float32
    B = x.shape[0]
    ngrp = B // G

    x4 = x.astype(BF).reshape(B, 16, 4, 64).transpose(2, 0, 1, 3)
    x4 = jnp.pad(x4, ((0, 0), (0, 0), (5, 3), (0, 0)))
    x4 = x4.reshape(4, ngrp, G * RPI, 64)
    x4 = jnp.pad(x4, ((0, 0), (0, 0), (0, SLOP), (0, 0)))

    band1 = _make_bands(c1w, 8, 1, 64)
    band2 = _make_bands(c2w, 16, 8, 32)
    band3 = _make_bands(c3w, 120, 16, 16, split_x=False)

    def tile_bias(b, width, split_x=True):
        t = jnp.repeat(b.reshape(-1), width // (2 if split_x else 1))
        if split_x:
            t = jnp.tile(t, (2,))
        return t.reshape(1, -1).astype(f32)

    a1 = tile_bias(c1b, 64)
    a2 = tile_bias(c2b, 32)
    a3 = tile_bias(c3b, 16, split_x=False)

    feat = pl.pallas_call(
        _conv_net_kernel,
        out_shape=jax.ShapeDtypeStruct((B * 16, 1920), BF),
        grid_spec=pltpu.PrefetchScalarGridSpec(
            num_scalar_prefetch=0,
            grid=(ngrp,),
            in_specs=[
                pl.BlockSpec((4, 1, G * RPI + SLOP, 64),
                             lambda i: (0, i, 0, 0)),
                pl.BlockSpec(band1.shape, lambda i: (0, 0, 0)),
                pl.BlockSpec(band2.shape, lambda i: (0, 0, 0)),
                pl.BlockSpec(band3.shape, lambda i: (0, 0, 0)),
                pl.BlockSpec(a1.shape, lambda i: (0, 0)),
                pl.BlockSpec(a2.shape, lambda i: (0, 0)),
                pl.BlockSpec(a3.shape, lambda i: (0, 0)),
            ],
            out_specs=pl.BlockSpec((G * 16, 1920), lambda i: (i, 0)),
            scratch_shapes=[
                pltpu.VMEM((G * RPI + SLOP, 256), BF),
                pltpu.VMEM((G * RPI + SLOP, 256), BF),
                pltpu.VMEM((G * RPI + SLOP, 256), BF),
            ],
        ),
        compiler_params=pltpu.CompilerParams(
            dimension_semantics=("parallel",)),
    )(x4, band1, band2, band3, a1, a2, a3)

    f2 = feat.reshape(B, 16 * 1920)
    w3 = fc1_wt.reshape(120, 256, 128)

    out = pl.pallas_call(
        _fc_kernel,
        out_shape=jax.ShapeDtypeStruct((B, 7), f32),
        grid_spec=pltpu.PrefetchScalarGridSpec(
            num_scalar_prefetch=0,
            grid=(16,),
            in_specs=[
                pl.BlockSpec((B, 1920), lambda j: (0, j)),
                pl.BlockSpec((120, 16, 128), lambda j: (0, j, 0)),
                pl.BlockSpec((1, 128), lambda j: (0, 0)),
                pl.BlockSpec((128, 7), lambda j: (0, 0)),
                pl.BlockSpec((1, 7), lambda j: (0, 0)),
            ],
            out_specs=pl.BlockSpec((B, 7), lambda j: (0, 0)),
            scratch_shapes=[pltpu.VMEM((B, 128), f32)],
        ),
        compiler_params=pltpu.CompilerParams(
            dimension_semantics=("arbitrary",)),
    )(f2, w3, fc1_b, fc2_wt, fc2_b)
    return out
